# baseline (speedup 1.0000x reference)
import functools

import jax
import jax.numpy as jnp
from jax.experimental import pallas as pl

_N = 200
_D = 4
_PIDS_NZ = (1, 2, 3, 4)
_BIG2 = 1e18


def _chamfer_kernel(x_ref, yt_ref, ip_ref, op_ref, nz_ref, z_ref, *, n_batches):
    i = pl.program_id(0)

    x = x_ref[0]
    yt = yt_ref[0]
    ip = ip_ref[0]
    op = op_ref[0]

    inv_b = jnp.float32(1.0 / n_batches)

    nx2 = jnp.sum(x * x, axis=1, keepdims=True)
    norm_x = jnp.sqrt(nx2)
    ny2 = jnp.sum(yt * yt, axis=0, keepdims=True)
    norm_y = jnp.sqrt(ny2)

    mz = op == 0
    n0 = jnp.maximum(1, jnp.sum(mz)).astype(jnp.float32)
    loss_zero = jnp.sum(jnp.where(mz, norm_y, 0.0)) / n0
    z_ref[...] = (loss_zero * inv_b).reshape(1, 1, 1)

    d2 = jnp.zeros((_N, _N), jnp.float32)
    for d in range(_D):
        dd = x[:, d : d + 1] - yt[d : d + 1, :]
        d2 = d2 + dd * dd

    valid = ip == op
    dm2 = jnp.where(valid, d2, jnp.float32(_BIG2))
    rmin = jnp.sqrt(jnp.min(dm2, axis=1, keepdims=True))
    cmin = jnp.sqrt(jnp.min(dm2, axis=0, keepdims=True))

    loss_nz = jnp.float32(0.0)
    for p in _PIDS_NZ:
        mx = ip == p
        my = op == p
        nin = jnp.sum(mx)
        nout = jnp.sum(my)
        ninp = jnp.maximum(1, nin).astype(jnp.float32)
        noutp = jnp.maximum(1, nout).astype(jnp.float32)
        loss_a = jnp.sum(jnp.where(mx, norm_x, 0.0)) / ninp
        loss_b = jnp.sum(jnp.where(my, norm_y, 0.0)) / noutp
        loss_c = 0.5 * (
            jnp.sum(jnp.where(mx, rmin, 0.0)) / noutp
            + jnp.sum(jnp.where(my, cmin, 0.0)) / ninp
        )
        loss_p = jnp.where(nout == 0, loss_a, jnp.where(nin == 0, loss_b, loss_c))
        loss_nz = loss_nz + loss_p

    @pl.when(i == 0)
    def _():
        nz_ref[...] = jnp.zeros((1, 1), jnp.float32)

    nz_ref[...] += (loss_nz * inv_b).reshape(1, 1)


def kernel(target, reco, in_pid, out_pid):
    b, n, d = target.shape
    yt = jnp.transpose(reco, (0, 2, 1))
    ip3 = in_pid.reshape(b, n, 1)
    op3 = out_pid.reshape(b, 1, n)

    nz, z = pl.pallas_call(
        functools.partial(_chamfer_kernel, n_batches=b),
        grid=(b,),
        in_specs=[
            pl.BlockSpec((1, n, d), lambda i: (i, 0, 0)),
            pl.BlockSpec((1, d, n), lambda i: (i, 0, 0)),
            pl.BlockSpec((1, n, 1), lambda i: (i, 0, 0)),
            pl.BlockSpec((1, 1, n), lambda i: (i, 0, 0)),
        ],
        out_specs=[
            pl.BlockSpec((1, 1), lambda i: (0, 0)),
            pl.BlockSpec((1, 1, 1), lambda i: (i, 0, 0)),
        ],
        out_shape=[
            jax.ShapeDtypeStruct((1, 1), jnp.float32),
            jax.ShapeDtypeStruct((b, 1, 1), jnp.float32),
        ],
    )(target, yt, ip3, op3)

    return nz.reshape(()), z.reshape(b)


# speedup vs baseline: 2.6380x; 2.6380x over previous
import functools

import jax
import jax.numpy as jnp
from jax.experimental import pallas as pl

_N = 200
_D = 4
_E = 8
_PIDS_NZ = (1, 2, 3, 4)
_BIG2 = 1e18


def _chamfer_kernel(x_ref, xt_ref, yt_ref, ip2_ref, op2_ref, ip3_ref, op3_ref,
                    nz_ref, z_ref, *, n_batches):
    i = pl.program_id(0)

    x = x_ref[...]
    xt = xt_ref[...]
    yt = yt_ref[...]
    ip2 = ip2_ref[...]
    op2 = op2_ref[...]
    ip3 = ip3_ref[...]
    op3 = op3_ref[...]

    inv_b = jnp.float32(1.0 / n_batches)

    nx2_row = jnp.sum(xt * xt, axis=1)
    ny2_row = jnp.sum(yt * yt, axis=1)
    norm_x = jnp.sqrt(nx2_row)
    norm_y = jnp.sqrt(ny2_row)

    mz = op2 == 0
    n0 = jnp.maximum(1, jnp.sum(mz, axis=1, keepdims=True)).astype(jnp.float32)
    loss_zero = jnp.sum(jnp.where(mz, norm_y, 0.0), axis=1, keepdims=True) / n0
    z_ref[...] = loss_zero * inv_b

    xy = jax.lax.dot_general(
        x, yt, (((2,), (1,)), ((0,), (0,))),
        preferred_element_type=jnp.float32,
    )
    nx2_col = jnp.sum(x * x, axis=2, keepdims=True)
    d2 = nx2_col + ny2_row[:, None, :] - 2.0 * xy

    valid = ip3 == op3
    dm2 = jnp.where(valid, d2, jnp.float32(_BIG2))
    rmin = jnp.sqrt(jnp.maximum(jnp.min(dm2, axis=2), 0.0))
    cmin = jnp.sqrt(jnp.maximum(jnp.min(dm2, axis=1), 0.0))

    loss_nz = jnp.zeros((_E, 1), jnp.float32)
    for p in _PIDS_NZ:
        mx = ip2 == p
        my = op2 == p
        nin = jnp.sum(mx, axis=1, keepdims=True)
        nout = jnp.sum(my, axis=1, keepdims=True)
        ninp = jnp.maximum(1, nin).astype(jnp.float32)
        noutp = jnp.maximum(1, nout).astype(jnp.float32)
        s_a = jnp.sum(jnp.where(mx, norm_x, 0.0), axis=1, keepdims=True)
        s_b = jnp.sum(jnp.where(my, norm_y, 0.0), axis=1, keepdims=True)
        s_cx = jnp.sum(jnp.where(mx, rmin, 0.0), axis=1, keepdims=True)
        s_cy = jnp.sum(jnp.where(my, cmin, 0.0), axis=1, keepdims=True)
        loss_a = s_a / ninp
        loss_b = s_b / noutp
        loss_c = 0.5 * (s_cx / noutp + s_cy / ninp)
        loss_p = jnp.where(nout == 0, loss_a, jnp.where(nin == 0, loss_b, loss_c))
        loss_nz = loss_nz + loss_p

    @pl.when(i == 0)
    def _():
        nz_ref[...] = jnp.zeros((1, 1), jnp.float32)

    nz_ref[...] += jnp.sum(loss_nz).reshape(1, 1) * inv_b


def kernel(target, reco, in_pid, out_pid):
    b, n, d = target.shape
    xt = jnp.transpose(target, (0, 2, 1))
    yt = jnp.transpose(reco, (0, 2, 1))
    ip3 = in_pid.reshape(b, n, 1)
    op3 = out_pid.reshape(b, 1, n)
    steps = b // _E

    nz, z = pl.pallas_call(
        functools.partial(_chamfer_kernel, n_batches=b),
        grid=(steps,),
        in_specs=[
            pl.BlockSpec((_E, n, d), lambda i: (i, 0, 0)),
            pl.BlockSpec((_E, d, n), lambda i: (i, 0, 0)),
            pl.BlockSpec((_E, d, n), lambda i: (i, 0, 0)),
            pl.BlockSpec((_E, n), lambda i: (i, 0)),
            pl.BlockSpec((_E, n), lambda i: (i, 0)),
            pl.BlockSpec((_E, n, 1), lambda i: (i, 0, 0)),
            pl.BlockSpec((_E, 1, n), lambda i: (i, 0, 0)),
        ],
        out_specs=[
            pl.BlockSpec((1, 1), lambda i: (0, 0)),
            pl.BlockSpec((_E, 1), lambda i: (i, 0)),
        ],
        out_shape=[
            jax.ShapeDtypeStruct((1, 1), jnp.float32),
            jax.ShapeDtypeStruct((b, 1), jnp.float32),
        ],
    )(target, xt, yt, in_pid, out_pid, ip3, op3)

    return nz.reshape(()), z.reshape(b)
